# baseline (speedup 1.0000x reference)
import jax
import jax.numpy as jnp
from jax.experimental import pallas as pl
from jax.experimental.pallas import tpu as pltpu

_CH = 1024


def _make_body(B, S, D, V, R):
    HPB = S // _CH
    N = B * HPB

    def body(x_ref, table_ref, out_ref, ibuf, obuf, pe_ref, in_sem, out_sem):
        def in_copy(c):
            slot = c % 2
            return pltpu.make_async_copy(
                x_ref.at[c // HPB, pl.ds((c % HPB) * _CH, _CH), :],
                ibuf.at[slot],
                in_sem.at[slot],
            )

        def out_copy(c):
            slot = c % 2
            return pltpu.make_async_copy(
                obuf.at[slot],
                out_ref.at[c // HPB, pl.ds((c % HPB) * _CH, _CH), :],
                out_sem.at[slot],
            )

        in_copy(0).start()

        i = jax.lax.broadcasted_iota(jnp.int32, (S, V), 0)
        k = jax.lax.broadcasted_iota(jnp.int32, (S, V), 1)
        rel = k - R
        counts = jnp.logical_and(rel >= -i, rel <= S - 1 - i).astype(jnp.float32)
        n_lo = jnp.maximum(i - (R - 1), 0).astype(jnp.float32)
        n_hi = jnp.maximum(S - i - R, 0).astype(jnp.float32)
        counts = jnp.where(k == 0, n_lo, counts)
        counts = jnp.where(k == V - 1, n_hi, counts)
        pe_ref[...] = jnp.dot(
            counts, table_ref[...], preferred_element_type=jnp.float32
        ) * (1.0 / S)

        for c in range(N):
            slot = c % 2
            if c + 1 < N:
                in_copy(c + 1).start()
            in_copy(c).wait()
            if c >= 2:
                out_copy(c - 2).wait()
            off = (c % HPB) * _CH
            obuf[slot, :, :] = ibuf[slot, :, :] + pe_ref[off : off + _CH, :]
            out_copy(c).start()
        out_copy(N - 2).wait()
        out_copy(N - 1).wait()

    return body


def kernel(x, table):
    B, S, D = x.shape
    V, _ = table.shape
    R = (V - 1) // 2
    return pl.pallas_call(
        _make_body(B, S, D, V, R),
        in_specs=[
            pl.BlockSpec(memory_space=pl.ANY),
            pl.BlockSpec(memory_space=pltpu.MemorySpace.VMEM),
        ],
        out_specs=pl.BlockSpec(memory_space=pl.ANY),
        out_shape=jax.ShapeDtypeStruct((B, S, D), x.dtype),
        scratch_shapes=[
            pltpu.VMEM((2, _CH, D), jnp.float32),
            pltpu.VMEM((2, _CH, D), jnp.float32),
            pltpu.VMEM((S, D), jnp.float32),
            pltpu.SemaphoreType.DMA((2,)),
            pltpu.SemaphoreType.DMA((2,)),
        ],
    )(x, table)


# speedup vs baseline: 1.1279x; 1.1279x over previous
import jax
import jax.numpy as jnp
from jax.experimental import pallas as pl
from jax.experimental.pallas import tpu as pltpu

def _make_body(B, S, D, V, R):
    def body(x_ref, table_ref, out_ref, buf, pe_ref, in_sem, out_sem):
        def in_copy(c):
            return pltpu.make_async_copy(
                x_ref.at[c], buf.at[c], in_sem.at[c]
            )

        def out_copy(c):
            return pltpu.make_async_copy(
                buf.at[c], out_ref.at[c], out_sem.at[c]
            )

        for c in range(B):
            in_copy(c).start()

        i = jax.lax.broadcasted_iota(jnp.int32, (S, V), 0)
        k = jax.lax.broadcasted_iota(jnp.int32, (S, V), 1)
        rel = k - R
        counts = jnp.logical_and(rel >= -i, rel <= S - 1 - i).astype(jnp.float32)
        n_lo = jnp.maximum(i - (R - 1), 0).astype(jnp.float32)
        n_hi = jnp.maximum(S - i - R, 0).astype(jnp.float32)
        counts = jnp.where(k == 0, n_lo, counts)
        counts = jnp.where(k == V - 1, n_hi, counts)
        pe_ref[...] = jnp.dot(
            counts, table_ref[...], preferred_element_type=jnp.float32
        ) * (1.0 / S)

        for c in range(B):
            in_copy(c).wait()
            buf[c, :, :] = buf[c, :, :] + pe_ref[...]
            out_copy(c).start()
        for c in range(B):
            out_copy(c).wait()

    return body


def kernel(x, table):
    B, S, D = x.shape
    V, _ = table.shape
    R = (V - 1) // 2
    return pl.pallas_call(
        _make_body(B, S, D, V, R),
        in_specs=[
            pl.BlockSpec(memory_space=pl.ANY),
            pl.BlockSpec(memory_space=pltpu.MemorySpace.VMEM),
        ],
        out_specs=pl.BlockSpec(memory_space=pl.ANY),
        out_shape=jax.ShapeDtypeStruct((B, S, D), x.dtype),
        scratch_shapes=[
            pltpu.VMEM((B, S, D), jnp.float32),
            pltpu.VMEM((S, D), jnp.float32),
            pltpu.SemaphoreType.DMA((B,)),
            pltpu.SemaphoreType.DMA((B,)),
        ],
    )(x, table)
